# Initial kernel scaffold; baseline (speedup 1.0000x reference)
import jax
import jax.numpy as jnp
from jax.experimental import pallas as pl


def kernel(x, table):
    raise NotImplementedError("write your pallas kernel here")


# speedup vs baseline: 11.3147x; 11.3147x over previous
import functools

import jax
import jax.numpy as jnp
from jax import lax
from jax.experimental import pallas as pl
from jax.experimental.pallas import tpu as pltpu
from jax.experimental.pallas import tpu_sc as plsc

NUM_EMBEDDINGS = 100000
EMB_DIM = 128
BATCH = 4096
SEQ = 200

HALF = SEQ // 2
NW = 32
B_PER_W = BATCH // NW
UNITS = 2 * B_PER_W
NCHUNK = EMB_DIM // 16


def _body(x_hbm, table_hbm, out_hbm, idx_v, buf0, buf1, out_v, sem0, sem1):
    nc = 2
    wid = lax.axis_index("s") * nc + lax.axis_index("c")
    ubase = wid * UNITS

    pltpu.sync_copy(x_hbm.at[pl.ds(ubase, UNITS)], idx_v)

    bufs = (buf0, buf1)
    sems = (sem0, sem1)

    def start(u, j):
        pltpu.async_copy(table_hbm.at[idx_v.at[u]], bufs[j], sems[j])

    def wait(j):
        pltpu.make_async_copy(table_hbm.at[idx_v.at[0]], bufs[j], sems[j]).wait()

    start(0, 0)

    def pair_body(p, carry):
        for j in range(2):
            u = p * 2 + j
            nxt = u + 1

            @pl.when(nxt < UNITS)
            def _():
                start(nxt, 1 - j)

            wait(j)

            def red(r, acc):
                return tuple(
                    acc[c] + bufs[j][r, pl.ds(c * 16, 16)] for c in range(NCHUNK)
                )

            acc = lax.fori_loop(
                0, HALF, red,
                tuple(jnp.zeros((16,), jnp.float32) for _ in range(NCHUNK)),
            )

            for c in range(NCHUNK):
                sl = pl.ds(c * 16, 16)
                if j == 0:
                    out_v[p, sl] = acc[c]
                else:
                    out_v[p, sl] = out_v[p, sl] + acc[c]
        return carry

    lax.fori_loop(0, B_PER_W, pair_body, 0)

    pltpu.sync_copy(out_v, out_hbm.at[pl.ds(wid * B_PER_W, B_PER_W)])


@functools.partial(
    pl.kernel,
    out_type=jax.ShapeDtypeStruct((BATCH, EMB_DIM), jnp.float32),
    mesh=plsc.VectorSubcoreMesh(core_axis_name="c", subcore_axis_name="s"),
    scratch_types=[
        pltpu.VMEM((UNITS, HALF), jnp.int32),
        pltpu.VMEM((HALF, EMB_DIM), jnp.float32),
        pltpu.VMEM((HALF, EMB_DIM), jnp.float32),
        pltpu.VMEM((B_PER_W, EMB_DIM), jnp.float32),
        pltpu.SemaphoreType.DMA,
        pltpu.SemaphoreType.DMA,
    ],
)
def _bow_sum(x_hbm, table_hbm, out_hbm, idx_v, buf0, buf1, out_v, sem0, sem1):
    _body(x_hbm, table_hbm, out_hbm, idx_v, buf0, buf1, out_v, sem0, sem1)


@jax.jit
def kernel(x, table):
    x2 = x.reshape(BATCH * 2, HALF).astype(jnp.int32)
    return _bow_sum(x2, table)


# speedup vs baseline: 16.1530x; 1.4276x over previous
import functools

import jax
import jax.numpy as jnp
from jax import lax
from jax.experimental import pallas as pl
from jax.experimental.pallas import tpu as pltpu
from jax.experimental.pallas import tpu_sc as plsc

NUM_EMBEDDINGS = 100000
EMB_DIM = 128
BATCH = 4096
SEQ = 200

NW = 32
B_PER_W = BATCH // NW
NCHUNK = EMB_DIM // 16
KFIRE = 8


def _body(x_hbm, table_hbm, out_hbm, idx_v, acc_v, sem):
    nc = 2
    wid = lax.axis_index("s") * nc + lax.axis_index("c")

    pltpu.sync_copy(x_hbm.at[wid], idx_v)

    zero = jnp.zeros((16,), jnp.float32)

    def zbody(p, carry):
        for c in range(NCHUNK):
            acc_v[p, pl.ds(c * 16, 16)] = zero
        return carry

    lax.fori_loop(0, B_PER_W, zbody, 0)

    def start(t):
        pltpu.async_copy(table_hbm.at[idx_v.at[t]], acc_v, sem, add=True)

    def wait():
        pltpu.make_async_copy(table_hbm.at[idx_v.at[0]], acc_v, sem).wait()

    def fire_drain(g, carry):
        for j in range(KFIRE):
            start(g * KFIRE + j)
        for j in range(KFIRE):
            wait()
        return carry

    lax.fori_loop(0, SEQ // KFIRE, fire_drain, 0)

    pltpu.sync_copy(acc_v, out_hbm.at[pl.ds(wid * B_PER_W, B_PER_W)])


@functools.partial(
    pl.kernel,
    out_type=jax.ShapeDtypeStruct((BATCH, EMB_DIM), jnp.float32),
    mesh=plsc.VectorSubcoreMesh(core_axis_name="c", subcore_axis_name="s"),
    scratch_types=[
        pltpu.VMEM((SEQ, B_PER_W), jnp.int32),
        pltpu.VMEM((B_PER_W, EMB_DIM), jnp.float32),
        pltpu.SemaphoreType.DMA,
    ],
)
def _bow_sum(x_hbm, table_hbm, out_hbm, idx_v, acc_v, sem):
    _body(x_hbm, table_hbm, out_hbm, idx_v, acc_v, sem)


@jax.jit
def kernel(x, table):
    xw = x.astype(jnp.int32).reshape(NW, B_PER_W, SEQ).transpose(0, 2, 1)
    return _bow_sum(xw, table)


# speedup vs baseline: 17.0263x; 1.0541x over previous
import functools

import jax
import jax.numpy as jnp
from jax import lax
from jax.experimental import pallas as pl
from jax.experimental.pallas import tpu as pltpu
from jax.experimental.pallas import tpu_sc as plsc

NUM_EMBEDDINGS = 100000
EMB_DIM = 128
BATCH = 4096
SEQ = 200

NW = 32
B_PER_W = BATCH // NW
NCHUNK = EMB_DIM // 16
KFIRE = 8


def _body(x_hbm, table_hbm, out_hbm, idx_v, acc0, acc1, sem):
    nc = 2
    wid = lax.axis_index("s") * nc + lax.axis_index("c")

    pltpu.sync_copy(x_hbm.at[wid], idx_v)

    zero = jnp.zeros((16,), jnp.float32)
    accs = (acc0, acc1)

    def zbody(p, carry):
        for c in range(NCHUNK):
            sl = pl.ds(c * 16, 16)
            acc0[p, sl] = zero
            acc1[p, sl] = zero
        return carry

    lax.fori_loop(0, B_PER_W, zbody, 0)

    def start(t, j):
        pltpu.async_copy(table_hbm.at[idx_v.at[t]], accs[j % 2], sem, add=True)

    def wait():
        pltpu.make_async_copy(table_hbm.at[idx_v.at[0]], acc0, sem).wait()

    for j in range(KFIRE):
        start(j, j)

    def fire_drain(g, carry):
        base = (g + 1) * KFIRE
        for j in range(KFIRE):
            start(base + j, j)
        for j in range(KFIRE):
            wait()
        return carry

    lax.fori_loop(0, SEQ // KFIRE - 1, fire_drain, 0)

    for j in range(KFIRE):
        wait()

    def mbody(p, carry):
        for c in range(NCHUNK):
            sl = pl.ds(c * 16, 16)
            acc0[p, sl] = acc0[p, sl] + acc1[p, sl]
        return carry

    lax.fori_loop(0, B_PER_W, mbody, 0)

    pltpu.sync_copy(acc0, out_hbm.at[pl.ds(wid * B_PER_W, B_PER_W)])


@functools.partial(
    pl.kernel,
    out_type=jax.ShapeDtypeStruct((BATCH, EMB_DIM), jnp.float32),
    mesh=plsc.VectorSubcoreMesh(core_axis_name="c", subcore_axis_name="s"),
    scratch_types=[
        pltpu.VMEM((SEQ, B_PER_W), jnp.int32),
        pltpu.VMEM((B_PER_W, EMB_DIM), jnp.float32),
        pltpu.VMEM((B_PER_W, EMB_DIM), jnp.float32),
        pltpu.SemaphoreType.DMA,
    ],
)
def _bow_sum(x_hbm, table_hbm, out_hbm, idx_v, acc0, acc1, sem):
    _body(x_hbm, table_hbm, out_hbm, idx_v, acc0, acc1, sem)


@jax.jit
def kernel(x, table):
    xw = x.astype(jnp.int32).reshape(NW, B_PER_W, SEQ).transpose(0, 2, 1)
    return _bow_sum(xw, table)
